# Initial kernel scaffold; baseline (speedup 1.0000x reference)
import functools

import jax
import jax.numpy as jnp
from jax import lax
from jax.experimental import pallas as pl
from jax.experimental.pallas import tpu as pltpu
from jax.experimental.pallas import tpu_sc as plsc

N = 10000
E = 320000
D = 128

NC = 2
NS = 16
EDGES_PER_TILE = E // (NC * NS)
CHUNK = 400
NCHUNKS = EDGES_PER_TILE // CHUNK

_mesh = plsc.VectorSubcoreMesh(core_axis_name="c", subcore_axis_name="s")


@functools.partial(
    pl.kernel,
    mesh=_mesh,
    out_type=jax.ShapeDtypeStruct((NC, N), jnp.float32),
    scratch_types=[
        pltpu.VMEM((CHUNK,), jnp.int32),
        pltpu.VMEM((CHUNK,), jnp.float32),
        pltpu.VMEM_SHARED((N,), jnp.float32),
    ],
)
def _deg_kernel(dst_hbm, z_hbm, out_hbm, dst_v, ones_v, acc_sh):
    c = lax.axis_index("c")
    s = lax.axis_index("s")

    @pl.when(s == 0)
    def _():
        pltpu.sync_copy(z_hbm, acc_sh)

    def fill_ones(j, carry):
        ones_v[pl.ds(j * 16, 16)] = jnp.ones((16,), jnp.float32)
        return carry

    lax.fori_loop(0, CHUNK // 16, fill_ones, 0)
    plsc.subcore_barrier()

    base0 = (c * NS + s) * EDGES_PER_TILE

    def chunk(k, carry):
        base = base0 + k * CHUNK
        pltpu.sync_copy(dst_hbm.at[pl.ds(base, CHUNK)], dst_v)
        pltpu.sync_copy(ones_v, acc_sh.at[dst_v], add=True)
        return carry

    lax.fori_loop(0, NCHUNKS, chunk, 0)
    plsc.subcore_barrier()

    @pl.when(s == 0)
    def _():
        pltpu.sync_copy(acc_sh, out_hbm.at[c])


@functools.partial(
    pl.kernel,
    mesh=_mesh,
    out_type=jax.ShapeDtypeStruct((NC, N, D), jnp.float32),
    scratch_types=[
        pltpu.VMEM((CHUNK,), jnp.int32),
        pltpu.VMEM((CHUNK,), jnp.int32),
        pltpu.VMEM((CHUNK, D), jnp.float32),
        pltpu.VMEM_SHARED((N, D), jnp.float32),
        pltpu.SemaphoreType.DMA,
    ],
)
def _prop_kernel(g_hbm, src_hbm, dst_hbm, z_hbm, out_hbm,
                 src_v, dst_v, rows_v, acc_sh, sem):
    c = lax.axis_index("c")
    s = lax.axis_index("s")

    @pl.when(s == 0)
    def _():
        pltpu.sync_copy(z_hbm, acc_sh)

    plsc.subcore_barrier()

    base0 = (c * NS + s) * EDGES_PER_TILE

    def chunk(k, carry):
        base = base0 + k * CHUNK
        pltpu.sync_copy(src_hbm.at[pl.ds(base, CHUNK)], src_v)
        pltpu.sync_copy(dst_hbm.at[pl.ds(base, CHUNK)], dst_v)
        pltpu.async_copy(g_hbm.at[src_v], rows_v, sem).wait()
        pltpu.sync_copy(rows_v, acc_sh.at[dst_v], add=True)
        return carry

    lax.fori_loop(0, NCHUNKS, chunk, 0)
    plsc.subcore_barrier()

    @pl.when(s == 0)
    def _():
        pltpu.sync_copy(acc_sh, out_hbm.at[c])


@functools.partial(
    pl.kernel,
    mesh=_mesh,
    out_type=jax.ShapeDtypeStruct((NC, N), jnp.float32),
    scratch_types=[
        pltpu.VMEM((CHUNK,), jnp.int32),
        pltpu.VMEM((CHUNK,), jnp.int32),
        pltpu.VMEM((CHUNK,), jnp.float32),
        pltpu.VMEM((N,), jnp.float32),
        pltpu.VMEM_SHARED((N,), jnp.float32),
    ],
)
def _prop1_kernel(g_hbm, src_hbm, dst_hbm, z_hbm, out_hbm,
                  src_v, dst_v, vals_v, g_v, acc_sh):
    c = lax.axis_index("c")
    s = lax.axis_index("s")

    @pl.when(s == 0)
    def _():
        pltpu.sync_copy(z_hbm, acc_sh)

    pltpu.sync_copy(g_hbm, g_v)
    plsc.subcore_barrier()

    base0 = (c * NS + s) * EDGES_PER_TILE

    def chunk(k, carry):
        base = base0 + k * CHUNK
        pltpu.sync_copy(src_hbm.at[pl.ds(base, CHUNK)], src_v)
        pltpu.sync_copy(dst_hbm.at[pl.ds(base, CHUNK)], dst_v)

        def gather16(j, inner):
            idx = src_v[pl.ds(j * 16, 16)]
            vals_v[pl.ds(j * 16, 16)] = plsc.load_gather(g_v, [idx])
            return inner

        lax.fori_loop(0, CHUNK // 16, gather16, 0)
        pltpu.sync_copy(vals_v, acc_sh.at[dst_v], add=True)
        return carry

    lax.fori_loop(0, NCHUNKS, chunk, 0)
    plsc.subcore_barrier()

    @pl.when(s == 0)
    def _():
        pltpu.sync_copy(acc_sh, out_hbm.at[c])


ROWS = 1000


def _tc1_body(degp_ref, x_ref, w_ref, g_ref, dinv_ref):
    d = degp_ref[0] + degp_ref[1] + 1.0
    dinv = lax.rsqrt(d)
    dinv_ref[...] = dinv
    h = jnp.dot(x_ref[...], w_ref[...], preferred_element_type=jnp.float32)
    g_ref[...] = h * dinv


def _tc_layer_body(p_ref, g_ref, dinv_ref, b_ref, w_ref, out_ref):
    s = p_ref[0] + p_ref[1] + g_ref[...]
    a = jnp.maximum(s * dinv_ref[...] + b_ref[...], 0.0)
    h = jnp.dot(a, w_ref[...], preferred_element_type=jnp.float32)
    out_ref[...] = h * dinv_ref[...]


def _tc_out_body(p_ref, g_ref, dinv_ref, b_ref, out_ref):
    out_ref[...] = (p_ref[0] + p_ref[1] + g_ref[...]) * dinv_ref[...] + b_ref[...]


def _tc1(degp, x, w0):
    return pl.pallas_call(
        _tc1_body,
        grid=(N // ROWS,),
        in_specs=[
            pl.BlockSpec((NC, ROWS, 1), lambda i: (0, i, 0)),
            pl.BlockSpec((ROWS, D), lambda i: (i, 0)),
            pl.BlockSpec((D, D), lambda i: (0, 0)),
        ],
        out_specs=[
            pl.BlockSpec((ROWS, D), lambda i: (i, 0)),
            pl.BlockSpec((ROWS, 1), lambda i: (i, 0)),
        ],
        out_shape=[
            jax.ShapeDtypeStruct((N, D), jnp.float32),
            jax.ShapeDtypeStruct((N, 1), jnp.float32),
        ],
    )(degp, x, w0)


def _tc_layer(p, g, dinv, b, w):
    dout = w.shape[1]
    return pl.pallas_call(
        _tc_layer_body,
        grid=(N // ROWS,),
        in_specs=[
            pl.BlockSpec((NC, ROWS, D), lambda i: (0, i, 0)),
            pl.BlockSpec((ROWS, D), lambda i: (i, 0)),
            pl.BlockSpec((ROWS, 1), lambda i: (i, 0)),
            pl.BlockSpec((1, D), lambda i: (0, 0)),
            pl.BlockSpec((D, dout), lambda i: (0, 0)),
        ],
        out_specs=pl.BlockSpec((ROWS, dout), lambda i: (i, 0)),
        out_shape=jax.ShapeDtypeStruct((N, dout), jnp.float32),
    )(p, g, dinv, b, w)


def _tc_out(p, g2, dinv, b2):
    return pl.pallas_call(
        _tc_out_body,
        grid=(N // ROWS,),
        in_specs=[
            pl.BlockSpec((NC, ROWS, 1), lambda i: (0, i, 0)),
            pl.BlockSpec((ROWS, 1), lambda i: (i, 0)),
            pl.BlockSpec((ROWS, 1), lambda i: (i, 0)),
            pl.BlockSpec((1, 1), lambda i: (0, 0)),
        ],
        out_specs=pl.BlockSpec((ROWS, 1), lambda i: (i, 0)),
        out_shape=jax.ShapeDtypeStruct((N, 1), jnp.float32),
    )(p, g2, dinv, b2)


def kernel(x, edge_index, W0, b0, W1, b1, W2, b2):
    src = edge_index[0].astype(jnp.int32)
    dst = edge_index[1].astype(jnp.int32)
    zN = jnp.zeros((N,), jnp.float32)
    zND = jnp.zeros((N, D), jnp.float32)

    degp = _deg_kernel(dst, zN)
    g0, dinv = _tc1(degp.reshape(NC, N, 1), x, W0)
    p0 = _prop_kernel(g0, src, dst, zND)
    g1 = _tc_layer(p0, g0, dinv, b0.reshape(1, D), W1)
    p1 = _prop_kernel(g1, src, dst, zND)
    g2 = _tc_layer(p1, g1, dinv, b1.reshape(1, D), W2)
    p2 = _prop1_kernel(g2.reshape(N), src, dst, zN)
    out = _tc_out(p2.reshape(NC, N, 1), g2, dinv, b2.reshape(1, 1))
    return out


# speedup vs baseline: 20.3241x; 20.3241x over previous
import functools

import jax
import jax.numpy as jnp
from jax import lax
from jax.experimental import pallas as pl
from jax.experimental.pallas import tpu as pltpu
from jax.experimental.pallas import tpu_sc as plsc

N = 10000
E = 320000
D = 128

NC = 2
NS = 16
EDGES_PER_TILE = E // (NC * NS)
CHUNK = 200
NCHUNKS = EDGES_PER_TILE // CHUNK
DEG_CHUNK = 400
DEG_NCHUNKS = EDGES_PER_TILE // DEG_CHUNK

_mesh = plsc.VectorSubcoreMesh(core_axis_name="c", subcore_axis_name="s")


@functools.partial(
    pl.kernel,
    mesh=_mesh,
    out_type=jax.ShapeDtypeStruct((NC, N), jnp.float32),
    scratch_types=[
        pltpu.VMEM((DEG_CHUNK,), jnp.int32),
        pltpu.VMEM((DEG_CHUNK,), jnp.float32),
        pltpu.VMEM_SHARED((N,), jnp.float32),
    ],
)
def _deg_kernel(dst_hbm, z_hbm, out_hbm, dst_v, ones_v, acc_sh):
    c = lax.axis_index("c")
    s = lax.axis_index("s")

    @pl.when(s == 0)
    def _():
        pltpu.sync_copy(z_hbm, acc_sh)

    def fill_ones(j, carry):
        ones_v[pl.ds(j * 16, 16)] = jnp.ones((16,), jnp.float32)
        return carry

    lax.fori_loop(0, DEG_CHUNK // 16, fill_ones, 0)
    plsc.subcore_barrier()

    base0 = (c * NS + s) * EDGES_PER_TILE

    def chunk(k, carry):
        base = base0 + k * DEG_CHUNK
        pltpu.sync_copy(dst_hbm.at[pl.ds(base, DEG_CHUNK)], dst_v)
        pltpu.sync_copy(ones_v, acc_sh.at[dst_v], add=True)
        return carry

    lax.fori_loop(0, DEG_NCHUNKS, chunk, 0)
    plsc.subcore_barrier()

    @pl.when(s == 0)
    def _():
        pltpu.sync_copy(acc_sh, out_hbm.at[c])


@functools.partial(
    pl.kernel,
    mesh=_mesh,
    out_type=jax.ShapeDtypeStruct((NC, N, D), jnp.float32),
    scratch_types=[
        pltpu.VMEM((CHUNK,), jnp.int32),
        pltpu.VMEM((CHUNK,), jnp.int32),
        pltpu.VMEM((CHUNK, D), jnp.float32),
        pltpu.VMEM_SHARED((N, D), jnp.float32),
        pltpu.SemaphoreType.DMA,
    ],
)
def _prop_kernel(g_hbm, src_hbm, dst_hbm, z_hbm, out_hbm,
                 src_v, dst_v, rows_v, acc_sh, sem):
    c = lax.axis_index("c")
    s = lax.axis_index("s")

    @pl.when(s == 0)
    def _():
        pltpu.sync_copy(z_hbm, acc_sh)

    plsc.subcore_barrier()

    base0 = (c * NS + s) * EDGES_PER_TILE

    def chunk(k, carry):
        base = base0 + k * CHUNK
        pltpu.sync_copy(src_hbm.at[pl.ds(base, CHUNK)], src_v)
        pltpu.sync_copy(dst_hbm.at[pl.ds(base, CHUNK)], dst_v)
        pltpu.async_copy(g_hbm.at[src_v], rows_v, sem).wait()
        pltpu.sync_copy(rows_v, acc_sh.at[dst_v], add=True)
        return carry

    lax.fori_loop(0, NCHUNKS, chunk, 0)
    plsc.subcore_barrier()

    @pl.when(s == 0)
    def _():
        pltpu.sync_copy(acc_sh, out_hbm.at[c])


@functools.partial(
    pl.kernel,
    mesh=_mesh,
    out_type=jax.ShapeDtypeStruct((NC, N), jnp.float32),
    scratch_types=[
        pltpu.VMEM((CHUNK,), jnp.int32),
        pltpu.VMEM((CHUNK,), jnp.int32),
        pltpu.VMEM((CHUNK,), jnp.float32),
        pltpu.VMEM_SHARED((N,), jnp.float32),
        pltpu.SemaphoreType.DMA,
    ],
)
def _prop1_kernel(g_hbm, src_hbm, dst_hbm, z_hbm, out_hbm,
                  src_v, dst_v, vals_v, acc_sh, sem):
    c = lax.axis_index("c")
    s = lax.axis_index("s")

    @pl.when(s == 0)
    def _():
        pltpu.sync_copy(z_hbm, acc_sh)

    plsc.subcore_barrier()

    base0 = (c * NS + s) * EDGES_PER_TILE

    def chunk(k, carry):
        base = base0 + k * CHUNK
        pltpu.sync_copy(src_hbm.at[pl.ds(base, CHUNK)], src_v)
        pltpu.sync_copy(dst_hbm.at[pl.ds(base, CHUNK)], dst_v)
        pltpu.async_copy(g_hbm.at[src_v], vals_v, sem).wait()
        pltpu.sync_copy(vals_v, acc_sh.at[dst_v], add=True)
        return carry

    lax.fori_loop(0, NCHUNKS, chunk, 0)
    plsc.subcore_barrier()

    @pl.when(s == 0)
    def _():
        pltpu.sync_copy(acc_sh, out_hbm.at[c])


ROWS = 1000


def _tc1_body(degp_ref, x_ref, w_ref, g_ref, dinv_ref):
    d = degp_ref[0] + degp_ref[1] + 1.0
    dinv = lax.rsqrt(d)
    dinv_ref[...] = dinv
    h = jnp.dot(x_ref[...], w_ref[...], preferred_element_type=jnp.float32)
    g_ref[...] = h * dinv


def _tc_layer_body(p_ref, g_ref, dinv_ref, b_ref, w_ref, out_ref):
    s = p_ref[0] + p_ref[1] + g_ref[...]
    a = jnp.maximum(s * dinv_ref[...] + b_ref[...], 0.0)
    h = jnp.dot(a, w_ref[...], preferred_element_type=jnp.float32)
    out_ref[...] = h * dinv_ref[...]


def _tc_out_body(p_ref, g_ref, dinv_ref, b_ref, out_ref):
    out_ref[...] = (p_ref[0] + p_ref[1] + g_ref[...]) * dinv_ref[...] + b_ref[...]


def _tc1(degp, x, w0):
    return pl.pallas_call(
        _tc1_body,
        grid=(N // ROWS,),
        in_specs=[
            pl.BlockSpec((NC, ROWS, 1), lambda i: (0, i, 0)),
            pl.BlockSpec((ROWS, D), lambda i: (i, 0)),
            pl.BlockSpec((D, D), lambda i: (0, 0)),
        ],
        out_specs=[
            pl.BlockSpec((ROWS, D), lambda i: (i, 0)),
            pl.BlockSpec((ROWS, 1), lambda i: (i, 0)),
        ],
        out_shape=[
            jax.ShapeDtypeStruct((N, D), jnp.float32),
            jax.ShapeDtypeStruct((N, 1), jnp.float32),
        ],
    )(degp, x, w0)


def _tc_layer(p, g, dinv, b, w):
    dout = w.shape[1]
    return pl.pallas_call(
        _tc_layer_body,
        grid=(N // ROWS,),
        in_specs=[
            pl.BlockSpec((NC, ROWS, D), lambda i: (0, i, 0)),
            pl.BlockSpec((ROWS, D), lambda i: (i, 0)),
            pl.BlockSpec((ROWS, 1), lambda i: (i, 0)),
            pl.BlockSpec((1, D), lambda i: (0, 0)),
            pl.BlockSpec((D, dout), lambda i: (0, 0)),
        ],
        out_specs=pl.BlockSpec((ROWS, dout), lambda i: (i, 0)),
        out_shape=jax.ShapeDtypeStruct((N, dout), jnp.float32),
    )(p, g, dinv, b, w)


def _tc_out(p, g2, dinv, b2):
    return pl.pallas_call(
        _tc_out_body,
        grid=(N // ROWS,),
        in_specs=[
            pl.BlockSpec((NC, ROWS, 1), lambda i: (0, i, 0)),
            pl.BlockSpec((ROWS, 1), lambda i: (i, 0)),
            pl.BlockSpec((ROWS, 1), lambda i: (i, 0)),
            pl.BlockSpec((1, 1), lambda i: (0, 0)),
        ],
        out_specs=pl.BlockSpec((ROWS, 1), lambda i: (i, 0)),
        out_shape=jax.ShapeDtypeStruct((N, 1), jnp.float32),
    )(p, g2, dinv, b2)


def kernel(x, edge_index, W0, b0, W1, b1, W2, b2):
    src = edge_index[0].astype(jnp.int32)
    dst = edge_index[1].astype(jnp.int32)
    zN = jnp.zeros((N,), jnp.float32)
    zND = jnp.zeros((N, D), jnp.float32)

    degp = _deg_kernel(dst, zN)
    g0, dinv = _tc1(degp.reshape(NC, N, 1), x, W0)
    p0 = _prop_kernel(g0, src, dst, zND)
    g1 = _tc_layer(p0, g0, dinv, b0.reshape(1, D), W1)
    p1 = _prop_kernel(g1, src, dst, zND)
    g2 = _tc_layer(p1, g1, dinv, b1.reshape(1, D), W2)
    p2 = _prop1_kernel(g2.reshape(N), src, dst, zN)
    out = _tc_out(p2.reshape(NC, N, 1), g2, dinv, b2.reshape(1, 1))
    return out
